# baseline (speedup 1.0000x reference)
import jax
import jax.numpy as jnp
from jax.experimental import pallas as pl
from jax.experimental.pallas import tpu as pltpu

B, A = 256, 8
D = 512
K = 3
BB = 16
NB = B // BB
BI = BB * A
PCH = 2


def _fused_fwd(xp_ref, w2_ref, bcv_ref,
               wl1_ref, bl1_ref, wr1_ref,
               wl2_ref, bl2_ref, wr2_ref,
               wl3_ref, bl3_ref, wr3_ref,
               wc_ref, bc_ref, out_ref):
    f32 = jnp.float32

    def mm(a, b):
        return jax.lax.dot_general(a, b, (((1,), (0,)), ((), ())),
                                   preferred_element_type=f32)

    def mm_t(a, b):
        return jax.lax.dot_general(a, b, (((1,), (1,)), ((), ())),
                                   preferred_element_type=f32)

    bf16 = jnp.bfloat16

    def mm_bf(a, b):
        return jax.lax.dot_general(a.astype(bf16), b.astype(bf16),
                                   (((1,), (0,)), ((), ())),
                                   preferred_element_type=f32)

    def mm_t_bf(a, b):
        return jax.lax.dot_general(a.astype(bf16), b.astype(bf16),
                                   (((1,), (1,)), ((), ())),
                                   preferred_element_type=f32)

    w2 = w2_ref[...]
    bcv = bcv_ref[...]
    arr = xp_ref[...].reshape(BI * 8, 8, 64)
    sub = jax.lax.broadcasted_iota(jnp.int32, (BI * 8, 8, 64), 1)
    lg = jax.lax.broadcasted_iota(jnp.int32, (BI * 8, 8, 64), 2) // 8
    xt = arr
    for s in (4, 2, 1):
        bs = (sub // s) % 2
        bl = (lg // s) % 2
        a = jnp.roll(jnp.roll(xt, -s, axis=1), 8 * s, axis=2)
        if s == 4:
            xt = jnp.where(bs != bl, a, xt)
        else:
            b = jnp.roll(jnp.roll(xt, s, axis=1), -8 * s, axis=2)
            xt = jnp.where((bs == 0) & (bl == 1), a,
                           jnp.where((bs == 1) & (bl == 0), b, xt))
    xs = xt.reshape(BI * 64, 64)
    ipc = BI // PCH
    enc_parts = []
    for c in range(PCH):
        fc = jnp.maximum(mm_bf(xs[c * ipc * 64:(c + 1) * ipc * 64], w2) + bcv,
                         0.0)
        enc_parts.append(fc.reshape(ipc, 64, D).sum(axis=1))
    enc = jnp.concatenate(enc_parts, axis=0) * (1.0 / 64.0)

    nn = jnp.sqrt(jnp.sum(enc * enc, axis=1, keepdims=True))
    nrm = enc / (nn + 1e-12)
    sim = mm_t(nrm, nrm)
    row = jax.lax.broadcasted_iota(jnp.int32, (BI, BI), 0)
    col = jax.lax.broadcasted_iota(jnp.int32, (BI, BI), 1)
    same = (row // A) == (col // A)
    s = jnp.where(same, sim, jnp.float32(-1e9))
    m = jnp.zeros((BI, BI), f32)
    for _ in range(K):
        mx = jnp.max(s, axis=1, keepdims=True)
        hit = s >= mx
        first = jnp.min(jnp.where(hit, col, BI), axis=1, keepdims=True)
        oh = col == first
        m = m + oh.astype(f32)
        s = jnp.where(oh, jnp.float32(-2e9), s)
    m = m * (1.0 / K)

    h = enc
    layers = ((wl1_ref, bl1_ref, wr1_ref, True),
              (wl2_ref, bl2_ref, wr2_ref, True),
              (wl3_ref, bl3_ref, wr3_ref, False))
    for wl_ref, bl_ref, wr_ref, act in layers:
        agg = mm(m, h)
        h_new = (mm_t_bf(agg, wl_ref[...]) + bl_ref[...]
                 + mm_t_bf(h, wr_ref[...]))
        h = jnp.where(h_new >= 0, h_new, 0.2 * h_new) if act else h_new

    prow = jax.lax.broadcasted_iota(jnp.int32, (BB, BI), 0)
    pcol = jax.lax.broadcasted_iota(jnp.int32, (BB, BI), 1)
    pool = jnp.where(prow == pcol // A, 1.0 / A, 0.0).astype(f32)
    pooled = mm(pool, h)
    out_ref[...] = mm_t(pooled, wc_ref[...]) + bc_ref[...]


def kernel(x, Wconv, bconv, Wl1, bl1, Wr1, Wl2, bl2, Wr2, Wl3, bl3, Wr3, Wc, bc):
    w2 = Wconv.reshape(D, 64).T
    bcv = bconv.reshape(1, D)
    wc_pad = jnp.zeros((128, D), jnp.float32).at[:2, :].set(Wc)
    bc_pad = jnp.zeros((1, 128), jnp.float32).at[0, :2].set(bc)
    full = lambda shape: pl.BlockSpec(shape, lambda i: (0,) * len(shape))
    wspec = full((D, D))
    bspec = full((1, D))

    def run(xpart, nb):
        return pl.pallas_call(
            _fused_fwd,
            grid=(nb,),
            in_specs=[
                pl.BlockSpec((BI, 64, 64), lambda i: (i, 0, 0)),
                full((64, D)),
                full((1, D)),
                wspec, bspec, wspec,
                wspec, bspec, wspec,
                wspec, bspec, wspec,
                full((128, D)),
                full((1, 128)),
            ],
            out_specs=pl.BlockSpec((BB, 128), lambda i: (i, 0)),
            out_shape=jax.ShapeDtypeStruct((nb * BB, 128), jnp.float32),
            compiler_params=pltpu.CompilerParams(
                dimension_semantics=("parallel",)),
        )(xpart, w2, bcv,
          Wl1, bl1.reshape(1, D), Wr1,
          Wl2, bl2.reshape(1, D), Wr2,
          Wl3, bl3.reshape(1, D), Wr3,
          wc_pad, bc_pad)

    return run(x.reshape(B * A, 64, 64), NB)[:, :2]


# speedup vs baseline: 1.0070x; 1.0070x over previous
import jax
import jax.numpy as jnp
from jax.experimental import pallas as pl
from jax.experimental.pallas import tpu as pltpu

B, A = 256, 8
D = 512
K = 3
BB = 16
NB = B // BB
BI = BB * A
PCH = 4


def _fused_fwd(xp_ref, w2_ref, bcv_ref,
               wl1_ref, bl1_ref, wr1_ref,
               wl2_ref, bl2_ref, wr2_ref,
               wl3_ref, bl3_ref, wr3_ref,
               wc_ref, bc_ref, out_ref):
    f32 = jnp.float32

    def mm(a, b):
        return jax.lax.dot_general(a, b, (((1,), (0,)), ((), ())),
                                   preferred_element_type=f32)

    def mm_t(a, b):
        return jax.lax.dot_general(a, b, (((1,), (1,)), ((), ())),
                                   preferred_element_type=f32)

    bf16 = jnp.bfloat16

    def mm_bf(a, b):
        return jax.lax.dot_general(a.astype(bf16), b.astype(bf16),
                                   (((1,), (0,)), ((), ())),
                                   preferred_element_type=f32)

    def mm_t_bf(a, b):
        return jax.lax.dot_general(a.astype(bf16), b.astype(bf16),
                                   (((1,), (1,)), ((), ())),
                                   preferred_element_type=f32)

    w2 = w2_ref[...]
    bcv = bcv_ref[...]
    arr = xp_ref[...].reshape(BI * 8, 8, 64)
    sub = jax.lax.broadcasted_iota(jnp.int32, (BI * 8, 8, 64), 1)
    lg = jax.lax.broadcasted_iota(jnp.int32, (BI * 8, 8, 64), 2) // 8
    xt = arr
    for s in (4, 2, 1):
        bs = (sub // s) % 2
        bl = (lg // s) % 2
        a = jnp.roll(jnp.roll(xt, -s, axis=1), 8 * s, axis=2)
        if s == 4:
            xt = jnp.where(bs != bl, a, xt)
        else:
            b = jnp.roll(jnp.roll(xt, s, axis=1), -8 * s, axis=2)
            xt = jnp.where((bs == 0) & (bl == 1), a,
                           jnp.where((bs == 1) & (bl == 0), b, xt))
    xs = xt.reshape(BI * 64, 64)
    ipc = BI // PCH
    enc_parts = []
    for c in range(PCH):
        fc = jnp.maximum(mm_bf(xs[c * ipc * 64:(c + 1) * ipc * 64], w2) + bcv,
                         0.0)
        enc_parts.append(fc.reshape(ipc, 64, D).sum(axis=1))
    enc = jnp.concatenate(enc_parts, axis=0) * (1.0 / 64.0)

    nn = jnp.sqrt(jnp.sum(enc * enc, axis=1, keepdims=True))
    nrm = enc / (nn + 1e-12)
    sim = mm_t(nrm, nrm)
    row = jax.lax.broadcasted_iota(jnp.int32, (BI, BI), 0)
    col = jax.lax.broadcasted_iota(jnp.int32, (BI, BI), 1)
    same = (row // A) == (col // A)
    s = jnp.where(same, sim, jnp.float32(-1e9))
    m = jnp.zeros((BI, BI), f32)
    for _ in range(K):
        mx = jnp.max(s, axis=1, keepdims=True)
        hit = s >= mx
        first = jnp.min(jnp.where(hit, col, BI), axis=1, keepdims=True)
        oh = col == first
        m = m + oh.astype(f32)
        s = jnp.where(oh, jnp.float32(-2e9), s)
    m = m * (1.0 / K)

    h = enc
    layers = ((wl1_ref, bl1_ref, wr1_ref, True),
              (wl2_ref, bl2_ref, wr2_ref, True),
              (wl3_ref, bl3_ref, wr3_ref, False))
    for wl_ref, bl_ref, wr_ref, act in layers:
        agg = mm(m, h)
        h_new = (mm_t_bf(agg, wl_ref[...]) + bl_ref[...]
                 + mm_t_bf(h, wr_ref[...]))
        h = jnp.where(h_new >= 0, h_new, 0.2 * h_new) if act else h_new

    prow = jax.lax.broadcasted_iota(jnp.int32, (BB, BI), 0)
    pcol = jax.lax.broadcasted_iota(jnp.int32, (BB, BI), 1)
    pool = jnp.where(prow == pcol // A, 1.0 / A, 0.0).astype(f32)
    pooled = mm(pool, h)
    out_ref[...] = mm_t(pooled, wc_ref[...]) + bc_ref[...]


def kernel(x, Wconv, bconv, Wl1, bl1, Wr1, Wl2, bl2, Wr2, Wl3, bl3, Wr3, Wc, bc):
    w2 = Wconv.reshape(D, 64).T
    bcv = bconv.reshape(1, D)
    wc_pad = jnp.zeros((128, D), jnp.float32).at[:2, :].set(Wc)
    bc_pad = jnp.zeros((1, 128), jnp.float32).at[0, :2].set(bc)
    full = lambda shape: pl.BlockSpec(shape, lambda i: (0,) * len(shape))
    wspec = full((D, D))
    bspec = full((1, D))

    def run(xpart, nb):
        return pl.pallas_call(
            _fused_fwd,
            grid=(nb,),
            in_specs=[
                pl.BlockSpec((BI, 64, 64), lambda i: (i, 0, 0)),
                full((64, D)),
                full((1, D)),
                wspec, bspec, wspec,
                wspec, bspec, wspec,
                wspec, bspec, wspec,
                full((128, D)),
                full((1, 128)),
            ],
            out_specs=pl.BlockSpec((BB, 128), lambda i: (i, 0)),
            out_shape=jax.ShapeDtypeStruct((nb * BB, 128), jnp.float32),
            compiler_params=pltpu.CompilerParams(
                dimension_semantics=("parallel",)),
        )(xpart, w2, bcv,
          Wl1, bl1.reshape(1, D), Wr1,
          Wl2, bl2.reshape(1, D), Wr2,
          Wl3, bl3.reshape(1, D), Wr3,
          wc_pad, bc_pad)

    return run(x.reshape(B * A, 64, 64), NB)[:, :2]
